# baseline (speedup 1.0000x reference)
import functools

import jax
import jax.numpy as jnp
from jax import lax
from jax.experimental import pallas as pl
from jax.experimental.pallas import tpu as pltpu
from jax.experimental.pallas import tpu_sc as plsc

_B, _L, _D, _MAXS, _NSEG = 4, 4096, 768, 100, 50
_NT = 8
_TPT = _L // _NT
_TCH = 32
_NCH = _TPT // _TCH
_LN = 16
_GPC = _TCH // _LN
_KV = _D // _LN
_NROW = _NSEG + 1

_mesh = plsc.VectorSubcoreMesh(core_axis_name="c", subcore_axis_name="s")


@functools.partial(
    pl.kernel,
    out_type=jax.ShapeDtypeStruct((_NT, _B, _NROW, _D), jnp.float32),
    mesh=_mesh,
    scratch_types=[
        pltpu.VMEM((_TCH, _D), jnp.float32),
        pltpu.VMEM((_TCH, _D), jnp.float32),
        pltpu.VMEM((_TPT,), jnp.int32),
        pltpu.VMEM((_NROW, _D), jnp.float32),
        pltpu.SemaphoreType.DMA,
        pltpu.SemaphoreType.DMA,
    ],
    compiler_params=pltpu.CompilerParams(needs_layout_passes=False),
)
def _sc_segmax(wf_hbm, ids_hbm, part_hbm, x0_v, x1_v, ids_v, acc_v, sem0,
               sem1):
    cid = lax.axis_index("c")
    sid = lax.axis_index("s")
    wid = sid * 2 + cid
    b = wid // _NT
    e = wid % _NT
    t0 = e * _TPT

    pltpu.sync_copy(ids_hbm.at[b, pl.ds(t0, _TPT)], ids_v)

    neg_inf = jnp.full((_LN,), -jnp.inf, jnp.float32)
    bufs = (x0_v, x1_v)
    sems = (sem0, sem1)

    def _init_row(i, carry):
        for k in range(_KV):
            acc_v[i, pl.ds(k * _LN, _LN)] = neg_inf
        return carry

    lax.fori_loop(0, _NROW, _init_row, 0)

    def _start(c):
        return pltpu.async_copy(
            wf_hbm.at[b, pl.ds(t0 + c * _TCH, _TCH), :],
            bufs[c % 2], sems[c % 2])

    pending = _start(0)
    for c in range(_NCH):
        nxt = _start(c + 1) if c + 1 < _NCH else None
        pending.wait()
        x_v = bufs[c % 2]
        idg = [ids_v[pl.ds(c * _TCH + g * _LN, _LN)] for g in range(_GPC)]
        mn = jnp.min(idg[0])
        mx = jnp.max(idg[-1])

        def _seg(s, carry, idg=idg, x_v=x_v):
            sp = jnp.full((_LN,), s, jnp.int32)
            st = jnp.sum((idg[0] < sp).astype(jnp.int32))
            en = jnp.sum((idg[0] <= sp).astype(jnp.int32))
            for g in range(1, _GPC):
                st = st + jnp.sum((idg[g] < sp).astype(jnp.int32))
                en = en + jnp.sum((idg[g] <= sp).astype(jnp.int32))

            def _tok(t, accs, x_v=x_v):
                return tuple(
                    jnp.maximum(a, x_v[t, pl.ds(k * _LN, _LN)])
                    for k, a in enumerate(accs))

            accs = lax.fori_loop(st, en, _tok, (neg_inf,) * _KV)
            for k in range(_KV):
                col = pl.ds(k * _LN, _LN)
                acc_v[s, col] = jnp.maximum(acc_v[s, col], accs[k])
            return carry

        lax.fori_loop(mn, mx + 1, _seg, 0)
        pending = nxt

    pltpu.sync_copy(acc_v, part_hbm.at[e, b])


def _tc_body(p_ref, w_ref, b_ref, v_ref, o_ref):
    w = w_ref[...]
    bias = b_ref[...]
    m = p_ref[0, 0, 1:, :]
    for e in range(1, _NT):
        m = jnp.maximum(m, p_ref[e, 0, 1:, :])
    valid = v_ref[0] != 0
    m = jnp.where(valid, m, 0.0)
    y = lax.dot_general(m, w, (((1,), (1,)), ((), ())),
                        preferred_element_type=jnp.float32) + bias
    o_ref[0, :_NSEG, :] = y
    o_ref[0, _NSEG:, :] = jnp.broadcast_to(bias, (_MAXS - _NSEG, _D))


_tc_linear = pl.pallas_call(
    _tc_body,
    grid=(_B,),
    in_specs=[
        pl.BlockSpec((_NT, 1, _NROW, _D), lambda i: (0, i, 0, 0)),
        pl.BlockSpec((_D, _D), lambda i: (0, 0)),
        pl.BlockSpec((1, _D), lambda i: (0, 0)),
        pl.BlockSpec((1, _NSEG, 1), lambda i: (i, 0, 0)),
    ],
    out_specs=pl.BlockSpec((1, _MAXS, _D), lambda i: (i, 0, 0)),
    out_shape=jax.ShapeDtypeStruct((_B, _MAXS, _D), jnp.float32),
)


def kernel(word_feature, sentence_mask, device, W, b):
    ids = sentence_mask.reshape(_B, _L).astype(jnp.int32)
    part = _sc_segmax(word_feature, ids)
    bb = ids[:, -1]
    valid = (jnp.arange(1, _NSEG + 1)[None, :, None] <= bb[:, None, None])
    return _tc_linear(part, W, b.reshape(1, _D), valid.astype(jnp.int32))


# speedup vs baseline: 1.1185x; 1.1185x over previous
import functools

import jax
import jax.numpy as jnp
from jax import lax
from jax.experimental import pallas as pl
from jax.experimental.pallas import tpu as pltpu
from jax.experimental.pallas import tpu_sc as plsc

_B, _L, _D, _MAXS, _NSEG = 4, 4096, 768, 100, 50
_NT = 8
_TPT = _L // _NT
_TCH = 32
_NCH = _TPT // _TCH
_NPAIR = _NCH // 2
_LN = 16
_GPC = _TCH // _LN
_KV = _D // _LN
_NROW = _NSEG + 1

_mesh = plsc.VectorSubcoreMesh(core_axis_name="c", subcore_axis_name="s")


@functools.partial(
    pl.kernel,
    out_type=jax.ShapeDtypeStruct((_NT, _B, _NROW, _D), jnp.float32),
    mesh=_mesh,
    scratch_types=[
        pltpu.VMEM((_TCH, _D), jnp.float32),
        pltpu.VMEM((_TCH, _D), jnp.float32),
        pltpu.VMEM((_TPT,), jnp.int32),
        pltpu.VMEM((_NROW, _D), jnp.float32),
        pltpu.SemaphoreType.DMA,
        pltpu.SemaphoreType.DMA,
    ],
    compiler_params=pltpu.CompilerParams(needs_layout_passes=False),
)
def _sc_segmax(wf_hbm, ids_hbm, part_hbm, x0_v, x1_v, ids_v, acc_v, sem0,
               sem1):
    cid = lax.axis_index("c")
    sid = lax.axis_index("s")
    wid = sid * 2 + cid
    b = wid // _NT
    e = wid % _NT
    t0 = e * _TPT

    pltpu.sync_copy(ids_hbm.at[b, pl.ds(t0, _TPT)], ids_v)

    neg_inf = jnp.full((_LN,), -jnp.inf, jnp.float32)

    def _init_row(i, carry):
        for k in range(_KV):
            acc_v[i, pl.ds(k * _LN, _LN)] = neg_inf
        return carry

    lax.fori_loop(0, _NROW, _init_row, 0)

    def _src(c):
        return wf_hbm.at[b, pl.ds(t0 + c * _TCH, _TCH), :]

    def _compute(c, x_v):
        idg = [ids_v[pl.ds(c * _TCH + g * _LN, _LN)] for g in range(_GPC)]
        mn = jnp.min(idg[0])
        mx = jnp.max(idg[-1])

        def _seg(s, carry):
            sp = jnp.full((_LN,), s, jnp.int32)
            st = jnp.sum((idg[0] < sp).astype(jnp.int32))
            en = jnp.sum((idg[0] <= sp).astype(jnp.int32))
            for g in range(1, _GPC):
                st = st + jnp.sum((idg[g] < sp).astype(jnp.int32))
                en = en + jnp.sum((idg[g] <= sp).astype(jnp.int32))

            def _tok(t, accs):
                return tuple(
                    jnp.maximum(a, x_v[t, pl.ds(k * _LN, _LN)])
                    for k, a in enumerate(accs))

            accs = lax.fori_loop(st, en, _tok, (neg_inf,) * _KV)
            for k in range(_KV):
                col = pl.ds(k * _LN, _LN)
                acc_v[s, col] = jnp.maximum(acc_v[s, col], accs[k])
            return carry

        lax.fori_loop(mn, mx + 1, _seg, 0)

    pltpu.async_copy(_src(0), x0_v, sem0)
    pltpu.async_copy(_src(1), x1_v, sem1)

    def _pair(cp, carry):
        c0 = 2 * cp
        pltpu.make_async_copy(_src(c0), x0_v, sem0).wait()
        _compute(c0, x0_v)

        @pl.when(cp < _NPAIR - 1)
        def _():
            pltpu.async_copy(_src(c0 + 2), x0_v, sem0)

        pltpu.make_async_copy(_src(c0 + 1), x1_v, sem1).wait()
        _compute(c0 + 1, x1_v)

        @pl.when(cp < _NPAIR - 1)
        def _():
            pltpu.async_copy(_src(c0 + 3), x1_v, sem1)

        return carry

    lax.fori_loop(0, _NPAIR, _pair, 0)

    pltpu.sync_copy(acc_v, part_hbm.at[e, b])


def _tc_body(p_ref, w_ref, b_ref, ids_ref, o_ref):
    w = w_ref[...]
    bias = b_ref[...]
    m = p_ref[0, 0, 1:, :]
    for e in range(1, _NT):
        m = jnp.maximum(m, p_ref[e, 0, 1:, :])
    bb = ids_ref[pl.program_id(0), 127]
    row = lax.broadcasted_iota(jnp.int32, (_NSEG, 1), 0) + 1
    m = jnp.where(row <= bb, m, 0.0)
    y = lax.dot_general(m, w, (((1,), (1,)), ((), ())),
                        preferred_element_type=jnp.float32) + bias
    o_ref[0, :_NSEG, :] = y
    o_ref[0, _NSEG:, :] = jnp.broadcast_to(bias, (_MAXS - _NSEG, _D))


_tc_linear = pl.pallas_call(
    _tc_body,
    grid=(_B,),
    in_specs=[
        pl.BlockSpec((_NT, 1, _NROW, _D), lambda i: (0, i, 0, 0)),
        pl.BlockSpec((_D, _D), lambda i: (0, 0)),
        pl.BlockSpec((1, _D), lambda i: (0, 0)),
        pl.BlockSpec((_B, 128), lambda i: (0, _L // 128 - 1)),
    ],
    out_specs=pl.BlockSpec((1, _MAXS, _D), lambda i: (i, 0, 0)),
    out_shape=jax.ShapeDtypeStruct((_B, _MAXS, _D), jnp.float32),
)


def kernel(word_feature, sentence_mask, device, W, b):
    ids = sentence_mask.reshape(_B, _L).astype(jnp.int32)
    part = _sc_segmax(word_feature, ids)
    return _tc_linear(part, W, b.reshape(1, _D), ids)


# speedup vs baseline: 1.1228x; 1.0039x over previous
import functools

import jax
import jax.numpy as jnp
from jax import lax
from jax.experimental import pallas as pl
from jax.experimental.pallas import tpu as pltpu
from jax.experimental.pallas import tpu_sc as plsc

_B, _L, _D, _MAXS, _NSEG = 4, 4096, 768, 100, 50
_NT = 8
_TPT = _L // _NT
_TCH = 32
_NCH = _TPT // _TCH
_NPAIR = _NCH // 2
_LN = 16
_GPC = _TCH // _LN
_KV = _D // _LN
_NROW = _NSEG + 8

_mesh = plsc.VectorSubcoreMesh(core_axis_name="c", subcore_axis_name="s")


@functools.partial(
    pl.kernel,
    out_type=jax.ShapeDtypeStruct((_NT, _B, _NROW, _D), jnp.float32),
    mesh=_mesh,
    scratch_types=[
        pltpu.VMEM((_TCH, _D), jnp.float32),
        pltpu.VMEM((_TCH, _D), jnp.float32),
        pltpu.VMEM((_TPT,), jnp.int32),
        pltpu.VMEM((_NROW, _D), jnp.float32),
        pltpu.SemaphoreType.DMA,
        pltpu.SemaphoreType.DMA,
    ],
    compiler_params=pltpu.CompilerParams(needs_layout_passes=False),
)
def _sc_segmax(wf_hbm, ids_hbm, part_hbm, x0_v, x1_v, ids_v, acc_v, sem0,
               sem1):
    cid = lax.axis_index("c")
    sid = lax.axis_index("s")
    wid = sid * 2 + cid
    b = wid // _NT
    e = wid % _NT
    t0 = e * _TPT

    pltpu.sync_copy(ids_hbm.at[b, pl.ds(t0, _TPT)], ids_v)

    neg_inf = jnp.full((_LN,), -jnp.inf, jnp.float32)

    def _init_row(i, carry):
        for k in range(_KV):
            acc_v[i, pl.ds(k * _LN, _LN)] = neg_inf
        return carry

    lax.fori_loop(0, _NROW, _init_row, 0)

    def _src(c):
        return wf_hbm.at[b, pl.ds(t0 + c * _TCH, _TCH), :]

    def _compute(c, x_v):
        idg = [ids_v[pl.ds(c * _TCH + g * _LN, _LN)] for g in range(_GPC)]
        mn = jnp.min(idg[0])
        mx = jnp.max(idg[-1])

        def _seg(s, carry):
            sp = jnp.full((_LN,), s, jnp.int32)
            st = jnp.sum((idg[0] < sp).astype(jnp.int32))
            en = jnp.sum((idg[0] <= sp).astype(jnp.int32))
            for g in range(1, _GPC):
                st = st + jnp.sum((idg[g] < sp).astype(jnp.int32))
                en = en + jnp.sum((idg[g] <= sp).astype(jnp.int32))

            def _tok(t, accs):
                return tuple(
                    jnp.maximum(a, x_v[t, pl.ds(k * _LN, _LN)])
                    for k, a in enumerate(accs))

            accs = lax.fori_loop(st, en, _tok, (neg_inf,) * _KV)
            r = jnp.where(s == 0, _NSEG, s - 1)
            for k in range(_KV):
                col = pl.ds(k * _LN, _LN)
                acc_v[r, col] = jnp.maximum(acc_v[r, col], accs[k])
            return carry

        lax.fori_loop(mn, mx + 1, _seg, 0)

    pltpu.async_copy(_src(0), x0_v, sem0)
    pltpu.async_copy(_src(1), x1_v, sem1)

    def _pair(cp, carry):
        c0 = 2 * cp
        pltpu.make_async_copy(_src(c0), x0_v, sem0).wait()
        _compute(c0, x0_v)

        @pl.when(cp < _NPAIR - 1)
        def _():
            pltpu.async_copy(_src(c0 + 2), x0_v, sem0)

        pltpu.make_async_copy(_src(c0 + 1), x1_v, sem1).wait()
        _compute(c0 + 1, x1_v)

        @pl.when(cp < _NPAIR - 1)
        def _():
            pltpu.async_copy(_src(c0 + 3), x1_v, sem1)

        return carry

    lax.fori_loop(0, _NPAIR, _pair, 0)

    pltpu.sync_copy(acc_v, part_hbm.at[e, b])


def _tc_body(p_ref, w_ref, b_ref, ids_ref, o_ref):
    w = w_ref[...]
    bias = b_ref[...]
    pad = jnp.broadcast_to(bias, (_MAXS - _NSEG, _D))
    j = pl.program_id(0)
    row = lax.broadcasted_iota(jnp.int32, (_NSEG, 1), 0) + 1
    for t in range(2):
        m = p_ref[0, t, :_NSEG, :]
        for e in range(1, _NT):
            m = jnp.maximum(m, p_ref[e, t, :_NSEG, :])
        bb = ids_ref[2 * j + t, 127]
        m = jnp.where(row <= bb, m, 0.0)
        y = lax.dot_general(m, w, (((1,), (1,)), ((), ())),
                            preferred_element_type=jnp.float32) + bias
        o_ref[t * _MAXS:t * _MAXS + _NSEG, :] = y
        o_ref[t * _MAXS + _NSEG:(t + 1) * _MAXS, :] = pad


_tc_linear = pl.pallas_call(
    _tc_body,
    grid=(_B // 2,),
    in_specs=[
        pl.BlockSpec((_NT, 2, _NROW, _D), lambda i: (0, i, 0, 0)),
        pl.BlockSpec((_D, _D), lambda i: (0, 0)),
        pl.BlockSpec((1, _D), lambda i: (0, 0)),
        pl.BlockSpec((_B, 128), lambda i: (0, _L // 128 - 1)),
    ],
    out_specs=pl.BlockSpec((2 * _MAXS, _D), lambda i: (i, 0)),
    out_shape=jax.ShapeDtypeStruct((_B * _MAXS, _D), jnp.float32),
)


def kernel(word_feature, sentence_mask, device, W, b):
    ids = sentence_mask.reshape(_B, _L).astype(jnp.int32)
    part = _sc_segmax(word_feature, ids)
    out = _tc_linear(part, W, b.reshape(1, _D), ids)
    return out.reshape(_B, _MAXS, _D)


# speedup vs baseline: 1.1796x; 1.0506x over previous
import functools

import jax
import jax.numpy as jnp
from jax import lax
from jax.experimental import pallas as pl
from jax.experimental.pallas import tpu as pltpu
from jax.experimental.pallas import tpu_sc as plsc

_B, _L, _D, _MAXS, _NSEG = 4, 4096, 768, 100, 50
_NT = 8
_TPT = _L // _NT
_TCH = 32
_NCH = _TPT // _TCH
_NTRI = (_NCH - 1) // 3
_LN = 16
_GPC = _TCH // _LN
_KV = _D // _LN
_NROW = _NSEG + 8

_mesh = plsc.VectorSubcoreMesh(core_axis_name="c", subcore_axis_name="s")


@functools.partial(
    pl.kernel,
    out_type=jax.ShapeDtypeStruct((_NT, _B, _NROW, _D), jnp.float32),
    mesh=_mesh,
    scratch_types=[
        pltpu.VMEM((_TCH, _D), jnp.float32),
        pltpu.VMEM((_TCH, _D), jnp.float32),
        pltpu.VMEM((_TCH, _D), jnp.float32),
        pltpu.VMEM((_TPT,), jnp.int32),
        pltpu.VMEM((_NROW, _D), jnp.float32),
        pltpu.SemaphoreType.DMA,
        pltpu.SemaphoreType.DMA,
        pltpu.SemaphoreType.DMA,
    ],
    compiler_params=pltpu.CompilerParams(needs_layout_passes=False),
)
def _sc_segmax(wf_hbm, ids_hbm, part_hbm, x0_v, x1_v, x2_v, ids_v, acc_v,
               sem0, sem1, sem2):
    cid = lax.axis_index("c")
    sid = lax.axis_index("s")
    wid = sid * 2 + cid
    b = wid // _NT
    e = wid % _NT
    t0 = e * _TPT

    pltpu.sync_copy(ids_hbm.at[b, pl.ds(t0, _TPT)], ids_v)

    neg_inf = jnp.full((_LN,), -jnp.inf, jnp.float32)
    bufs = (x0_v, x1_v, x2_v)
    sems = (sem0, sem1, sem2)

    def _init_row(i, carry):
        for k in range(_KV):
            acc_v[i, pl.ds(k * _LN, _LN)] = neg_inf
        return carry

    lax.fori_loop(0, _NROW, _init_row, 0)

    def _src(c):
        return wf_hbm.at[b, pl.ds(t0 + c * _TCH, _TCH), :]

    def _wait(c, u):
        pltpu.make_async_copy(_src(c), bufs[u], sems[u]).wait()

    def _issue(c, u):
        pltpu.async_copy(_src(c), bufs[u], sems[u])

    def _compute(c, x_v):
        idg = [ids_v[pl.ds(c * _TCH + g * _LN, _LN)] for g in range(_GPC)]
        mn = jnp.min(idg[0])
        mx = jnp.max(idg[-1])

        def _seg(s, carry):
            sp = jnp.full((_LN,), s, jnp.int32)
            st = jnp.sum((idg[0] < sp).astype(jnp.int32))
            en = jnp.sum((idg[0] <= sp).astype(jnp.int32))
            for g in range(1, _GPC):
                st = st + jnp.sum((idg[g] < sp).astype(jnp.int32))
                en = en + jnp.sum((idg[g] <= sp).astype(jnp.int32))

            def _tok(t, accs):
                return tuple(
                    jnp.maximum(a, x_v[t, pl.ds(k * _LN, _LN)])
                    for k, a in enumerate(accs))

            accs = lax.fori_loop(st, en, _tok, (neg_inf,) * _KV)
            r = jnp.where(s == 0, _NSEG, s - 1)
            for k in range(_KV):
                col = pl.ds(k * _LN, _LN)
                acc_v[r, col] = jnp.maximum(acc_v[r, col], accs[k])
            return carry

        lax.fori_loop(mn, mx + 1, _seg, 0)

    _issue(0, 0)
    _issue(1, 1)
    _issue(2, 2)

    def _triple(ct, carry):
        c0 = 3 * ct
        for u in range(3):
            _wait(c0 + u, u)
            _compute(c0 + u, bufs[u])
            if u == 0:
                _issue(c0 + 3, 0)
            else:
                @pl.when(ct < _NTRI - 1)
                def _(u=u):
                    _issue(c0 + u + 3, u)

        return carry

    lax.fori_loop(0, _NTRI, _triple, 0)
    _wait(_NCH - 1, (_NCH - 1) % 3)
    _compute(_NCH - 1, bufs[(_NCH - 1) % 3])

    pltpu.sync_copy(acc_v, part_hbm.at[e, b])


def _tc_body(p_ref, w_ref, b_ref, ids_ref, o_ref):
    w = w_ref[...]
    bias = b_ref[...]
    pad = jnp.broadcast_to(bias, (_MAXS - _NSEG, _D))
    j = pl.program_id(0)
    row = lax.broadcasted_iota(jnp.int32, (_NSEG, 1), 0) + 1
    for t in range(2):
        m = p_ref[0, t, :_NSEG, :]
        for e in range(1, _NT):
            m = jnp.maximum(m, p_ref[e, t, :_NSEG, :])
        bb = ids_ref[2 * j + t, 127]
        m = jnp.where(row <= bb, m, 0.0)
        y = lax.dot_general(m, w, (((1,), (1,)), ((), ())),
                            preferred_element_type=jnp.float32) + bias
        o_ref[t * _MAXS:t * _MAXS + _NSEG, :] = y
        o_ref[t * _MAXS + _NSEG:(t + 1) * _MAXS, :] = pad


_tc_linear = pl.pallas_call(
    _tc_body,
    grid=(_B // 2,),
    in_specs=[
        pl.BlockSpec((_NT, 2, _NROW, _D), lambda i: (0, i, 0, 0)),
        pl.BlockSpec((_D, _D), lambda i: (0, 0)),
        pl.BlockSpec((1, _D), lambda i: (0, 0)),
        pl.BlockSpec((_B, 128), lambda i: (0, _L // 128 - 1)),
    ],
    out_specs=pl.BlockSpec((2 * _MAXS, _D), lambda i: (i, 0)),
    out_shape=jax.ShapeDtypeStruct((_B * _MAXS, _D), jnp.float32),
)


def kernel(word_feature, sentence_mask, device, W, b):
    ids = sentence_mask.reshape(_B, _L).astype(jnp.int32)
    part = _sc_segmax(word_feature, ids)
    out = _tc_linear(part, W, b.reshape(1, _D), ids)
    return out.reshape(_B, _MAXS, _D)
